# Initial kernel scaffold; baseline (speedup 1.0000x reference)
import jax
import jax.numpy as jnp
from jax.experimental import pallas as pl


def kernel(pos, h, batch, atom_embed, W_in, b_in, Wc1, bc1, Wc2, bc2, Wc3, bc3, Wo1, bo1, Wo2, bo2):
    raise NotImplementedError("write your pallas kernel here")


# speedup vs baseline: 6.2686x; 6.2686x over previous
import functools

import jax
import jax.numpy as jnp
from jax import lax
from jax.experimental import pallas as pl
from jax.experimental.pallas import tpu as pltpu
from jax.experimental.pallas import tpu_sc as plsc

N = 10000
G = 200
A = 128
H = 128

NW = 32
NPW = 320
NPAD = NW * NPW
ROWW = 136
ACCSZ = (G + 1) * ROWW
NCHUNK = NPW // 16
RSZ = (G + 8) * 8

_mesh = plsc.VectorSubcoreMesh(core_axis_name="c", subcore_axis_name="s")


@functools.partial(
    pl.kernel,
    mesh=_mesh,
    out_type=jax.ShapeDtypeStruct((2, ACCSZ), jnp.float32),
    scratch_types=[
        pltpu.VMEM((NPW,), jnp.int32),
        pltpu.VMEM((NPW,), jnp.int32),
        pltpu.VMEM((NPW,), jnp.float32),
        pltpu.VMEM((NPW,), jnp.float32),
        pltpu.VMEM((NPW,), jnp.float32),
        pltpu.VMEM((10, 128), jnp.int32),
        pltpu.VMEM((10, 128), jnp.float32),
        pltpu.VMEM_SHARED((ACCSZ,), jnp.float32),
    ],
)
def _reduce_k(batch_hbm, h_hbm, px_hbm, py_hbm, pz_hbm, zeros_hbm, out_hbm,
              bt_v, h_v, px_v, py_v, pz_v, idx_st, val_st, acc_sh):
    c = lax.axis_index("c")
    s = lax.axis_index("s")
    wid = s * 2 + c
    base = wid * NPW

    @pl.when(s == 0)
    def _():
        pltpu.sync_copy(zeros_hbm, acc_sh)
    plsc.subcore_barrier()

    pltpu.sync_copy(batch_hbm.at[pl.ds(base, NPW)], bt_v)
    pltpu.sync_copy(h_hbm.at[pl.ds(base, NPW)], h_v)
    pltpu.sync_copy(px_hbm.at[pl.ds(base, NPW)], px_v)
    pltpu.sync_copy(py_hbm.at[pl.ds(base, NPW)], py_v)
    pltpu.sync_copy(pz_hbm.at[pl.ds(base, NPW)], pz_v)

    ones = jnp.full((16,), 1.0, jnp.float32)

    def _put(sec, j, ivec, vvec):
        e0 = sec * NPW + j * 16
        idx_st[e0 // 128, pl.ds(e0 % 128, 16)] = ivec
        val_st[e0 // 128, pl.ds(e0 % 128, 16)] = vvec

    for j in range(NCHUNK):
        sl = pl.ds(j * 16, 16)
        brow = bt_v[sl] * ROWW
        _put(0, j, brow + h_v[sl], ones)
        _put(1, j, brow + 128, px_v[sl])
        _put(2, j, brow + 129, py_v[sl])
        _put(3, j, brow + 130, pz_v[sl])

    for rr in range(10):
        pltpu.sync_copy(val_st.at[rr], acc_sh.at[idx_st.at[rr]], add=True)

    plsc.subcore_barrier()

    @pl.when(s == 0)
    def _():
        pltpu.sync_copy(acc_sh, out_hbm.at[c])


def _chain_body(stats_ref, emb_ref, wpos_ref, winh_ref, bin_ref,
                wc1_ref, bc1_ref, wc2_ref, bc2_ref, wc3_ref, bc3_ref,
                wo1_ref, bo1_ref, wo2_ref, bo2_ref, r_ref):
    st = stats_ref[0] + stats_ref[1]
    hist = st[:G, :A]
    pos8 = st[:G, A:A + 8]
    cnt = jnp.sum(hist, axis=1, keepdims=True)
    inv = jnp.where(cnt > 0, 1.0 / cnt, 0.0)
    f32 = jnp.float32
    wemb = jnp.dot(emb_ref[...], winh_ref[...], preferred_element_type=f32)
    msum = (jnp.dot(hist, wemb, preferred_element_type=f32)
            + jnp.dot(pos8, wpos_ref[...], preferred_element_type=f32))
    x = msum * inv + bin_ref[...]
    for w_ref, b_ref in ((wc1_ref, bc1_ref), (wc2_ref, bc2_ref),
                         (wc3_ref, bc3_ref), (wo1_ref, bo1_ref)):
        x = jax.nn.silu(jnp.dot(x, w_ref[...], preferred_element_type=f32)
                        + b_ref[...])
    r = jnp.dot(x, wo2_ref[...], preferred_element_type=f32) + bo2_ref[...]
    r_ref[...] = jnp.concatenate([r, jnp.zeros((8, 8), f32)], axis=0)


_chain = pl.pallas_call(
    _chain_body,
    out_shape=jax.ShapeDtypeStruct((G + 8, 8), jnp.float32),
)


@functools.partial(
    pl.kernel,
    mesh=_mesh,
    out_type=jax.ShapeDtypeStruct((3 * N,), jnp.float32),
    scratch_types=[
        pltpu.VMEM((8, 128), jnp.int32),
        pltpu.VMEM((1024,), jnp.float32),
    ],
)
def _gather_k(r_hbm, gidx_hbm, out_hbm, idx_v, out_v):
    c = lax.axis_index("c")
    s = lax.axis_index("s")
    wid = s * 2 + c

    pltpu.sync_copy(gidx_hbm.at[wid], idx_v)
    for rr in range(8):
        pltpu.sync_copy(r_hbm.at[idx_v.at[rr]], out_v.at[pl.ds(rr * 128, 128)])

    @pl.when(wid != NW - 1)
    def _():
        pltpu.sync_copy(out_v.at[pl.ds(0, 3 * NPW)],
                        out_hbm.at[pl.ds(wid * 3 * NPW, 3 * NPW)])

    @pl.when(wid == NW - 1)
    def _():
        tail = 3 * N - 3 * (NW - 1) * NPW
        pltpu.sync_copy(out_v.at[pl.ds(0, tail)],
                        out_hbm.at[pl.ds(3 * (NW - 1) * NPW, tail)])


def kernel(pos, h, batch, atom_embed, W_in, b_in, Wc1, bc1, Wc2, bc2,
           Wc3, bc3, Wo1, bo1, Wo2, bo2):
    pad = NPAD - N
    batch_p = jnp.concatenate([batch.astype(jnp.int32),
                               jnp.full((pad,), G, jnp.int32)])
    h_p = jnp.concatenate([h.astype(jnp.int32), jnp.zeros((pad,), jnp.int32)])
    pos_p = jnp.concatenate([pos, jnp.zeros((pad, 3), jnp.float32)])
    zeros = jnp.zeros((ACCSZ,), jnp.float32)

    stats = _reduce_k(batch_p, h_p, pos_p[:, 0], pos_p[:, 1], pos_p[:, 2],
                      zeros)
    stats = stats.reshape(2, G + 1, ROWW)

    wpos = jnp.concatenate([W_in[:3], jnp.zeros((5, H), jnp.float32)])
    wo2p = jnp.concatenate([Wo2, jnp.zeros((H, 5), jnp.float32)], axis=1)
    bo2p = jnp.concatenate([bo2, jnp.zeros((5,), jnp.float32)])

    r = _chain(stats, atom_embed, wpos, W_in[3:], b_in.reshape(1, H),
               Wc1, bc1.reshape(1, H), Wc2, bc2.reshape(1, H),
               Wc3, bc3.reshape(1, H), Wo1, bo1.reshape(1, H),
               wo2p, bo2p.reshape(1, 8))

    gidx = (batch_p[:, None] * 8 + jnp.arange(3, dtype=jnp.int32)).reshape(NW, 3 * NPW)
    gidx = jnp.pad(gidx, ((0, 0), (0, 64))).reshape(NW, 8, 128)

    out = _gather_k(r.reshape(RSZ), gidx)
    return out.reshape(N, 3)


# speedup vs baseline: 6.2917x; 1.0037x over previous
import functools

import jax
import jax.numpy as jnp
from jax import lax
from jax.experimental import pallas as pl
from jax.experimental.pallas import tpu as pltpu
from jax.experimental.pallas import tpu_sc as plsc

N = 10000
G = 200
A = 128
H = 128

NW = 32
NPW = 320
NPAD = NW * NPW
ROWW = 136
ACCSZ = (G + 1) * ROWW
ACCPAD = 27392
ZSTRIPE = ACCPAD // 16
NCHUNK = NPW // 16
RSZ = (G + 8) * 8

_mesh = plsc.VectorSubcoreMesh(core_axis_name="c", subcore_axis_name="s")


@functools.partial(
    pl.kernel,
    mesh=_mesh,
    out_type=jax.ShapeDtypeStruct((2, ACCPAD), jnp.float32),
    scratch_types=[
        pltpu.VMEM((NPW,), jnp.int32),
        pltpu.VMEM((NPW,), jnp.int32),
        pltpu.VMEM((NPW,), jnp.float32),
        pltpu.VMEM((NPW,), jnp.float32),
        pltpu.VMEM((NPW,), jnp.float32),
        pltpu.VMEM((10, 128), jnp.int32),
        pltpu.VMEM((10, 128), jnp.float32),
        pltpu.VMEM_SHARED((ACCPAD,), jnp.float32),
        pltpu.SemaphoreType.DMA,
        pltpu.SemaphoreType.DMA,
        pltpu.SemaphoreType.DMA,
    ],
)
def _reduce_k(batch_hbm, h_hbm, px_hbm, py_hbm, pz_hbm, zeros_hbm, out_hbm,
              bt_v, h_v, px_v, py_v, pz_v, idx_st, val_st, acc_sh,
              sem_in, sem_z, sem_sc):
    c = lax.axis_index("c")
    s = lax.axis_index("s")
    wid = s * 2 + c
    base = wid * NPW

    in_cp = [
        pltpu.async_copy(batch_hbm.at[pl.ds(base, NPW)], bt_v, sem_in),
        pltpu.async_copy(h_hbm.at[pl.ds(base, NPW)], h_v, sem_in),
        pltpu.async_copy(px_hbm.at[pl.ds(base, NPW)], px_v, sem_in),
        pltpu.async_copy(py_hbm.at[pl.ds(base, NPW)], py_v, sem_in),
        pltpu.async_copy(pz_hbm.at[pl.ds(base, NPW)], pz_v, sem_in),
    ]
    @pl.when(s == 0)
    def _():
        pltpu.async_copy(zeros_hbm, acc_sh, sem_z)
    for cp in in_cp:
        cp.wait()

    ones = jnp.full((16,), 1.0, jnp.float32)

    def _put(sec, j, ivec, vvec):
        e0 = sec * NPW + j * 16
        idx_st[e0 // 128, pl.ds(e0 % 128, 16)] = ivec
        val_st[e0 // 128, pl.ds(e0 % 128, 16)] = vvec

    for j in range(NCHUNK):
        sl = pl.ds(j * 16, 16)
        brow = bt_v[sl] * ROWW
        _put(0, j, brow + h_v[sl], ones)
        _put(1, j, brow + 128, px_v[sl])
        _put(2, j, brow + 129, py_v[sl])
        _put(3, j, brow + 130, pz_v[sl])

    @pl.when(s == 0)
    def _():
        pltpu.make_async_copy(zeros_hbm, acc_sh, sem_z).wait()
    plsc.subcore_barrier()

    sc_cp = [pltpu.async_copy(val_st.at[rr], acc_sh.at[idx_st.at[rr]],
                              sem_sc, add=True) for rr in range(10)]
    for cp in sc_cp:
        cp.wait()

    plsc.subcore_barrier()

    @pl.when(s == 0)
    def _():
        pltpu.sync_copy(acc_sh, out_hbm.at[c])


def _chain_body(stats_ref, emb_ref, wpos_ref, winh_ref, bin_ref,
                wc1_ref, bc1_ref, wc2_ref, bc2_ref, wc3_ref, bc3_ref,
                wo1_ref, bo1_ref, wo2_ref, bo2_ref, r_ref):
    st = stats_ref[0] + stats_ref[1]
    hist = st[:G, :A]
    pos8 = st[:G, A:A + 8]
    cnt = jnp.sum(hist, axis=1, keepdims=True)
    inv = jnp.where(cnt > 0, 1.0 / cnt, 0.0)
    f32 = jnp.float32
    wemb = jnp.dot(emb_ref[...], winh_ref[...], preferred_element_type=f32)
    msum = (jnp.dot(hist, wemb, preferred_element_type=f32)
            + jnp.dot(pos8, wpos_ref[...], preferred_element_type=f32))
    x = msum * inv + bin_ref[...]
    for w_ref, b_ref in ((wc1_ref, bc1_ref), (wc2_ref, bc2_ref),
                         (wc3_ref, bc3_ref), (wo1_ref, bo1_ref)):
        x = jax.nn.silu(jnp.dot(x, w_ref[...], preferred_element_type=f32)
                        + b_ref[...])
    r = jnp.dot(x, wo2_ref[...], preferred_element_type=f32) + bo2_ref[...]
    r_ref[...] = jnp.concatenate([r, jnp.zeros((8, 8), f32)], axis=0)


_chain = pl.pallas_call(
    _chain_body,
    out_shape=jax.ShapeDtypeStruct((G + 8, 8), jnp.float32),
)


@functools.partial(
    pl.kernel,
    mesh=_mesh,
    out_type=jax.ShapeDtypeStruct((3 * N,), jnp.float32),
    scratch_types=[
        pltpu.VMEM((8, 128), jnp.int32),
        pltpu.VMEM((1024,), jnp.float32),
        pltpu.SemaphoreType.DMA,
    ],
)
def _gather_k(r_hbm, gidx_hbm, out_hbm, idx_v, out_v, sem):
    c = lax.axis_index("c")
    s = lax.axis_index("s")
    wid = s * 2 + c

    pltpu.sync_copy(gidx_hbm.at[wid], idx_v)
    g_cp = [pltpu.async_copy(r_hbm.at[idx_v.at[rr]],
                             out_v.at[pl.ds(rr * 128, 128)], sem)
            for rr in range(8)]
    for cp in g_cp:
        cp.wait()

    @pl.when(wid != NW - 1)
    def _():
        pltpu.sync_copy(out_v.at[pl.ds(0, 3 * NPW)],
                        out_hbm.at[pl.ds(wid * 3 * NPW, 3 * NPW)])

    @pl.when(wid == NW - 1)
    def _():
        tail = 3 * N - 3 * (NW - 1) * NPW
        pltpu.sync_copy(out_v.at[pl.ds(0, tail)],
                        out_hbm.at[pl.ds(3 * (NW - 1) * NPW, tail)])


def kernel(pos, h, batch, atom_embed, W_in, b_in, Wc1, bc1, Wc2, bc2,
           Wc3, bc3, Wo1, bo1, Wo2, bo2):
    pad = NPAD - N
    batch_p = jnp.concatenate([batch.astype(jnp.int32),
                               jnp.full((pad,), G, jnp.int32)])
    h_p = jnp.concatenate([h.astype(jnp.int32), jnp.zeros((pad,), jnp.int32)])
    pos_p = jnp.concatenate([pos, jnp.zeros((pad, 3), jnp.float32)])
    zeros = jnp.zeros((ACCPAD,), jnp.float32)

    stats = _reduce_k(batch_p, h_p, pos_p[:, 0], pos_p[:, 1], pos_p[:, 2],
                      zeros)
    stats = stats[:, :ACCSZ].reshape(2, G + 1, ROWW)

    wpos = jnp.concatenate([W_in[:3], jnp.zeros((5, H), jnp.float32)])
    wo2p = jnp.concatenate([Wo2, jnp.zeros((H, 5), jnp.float32)], axis=1)
    bo2p = jnp.concatenate([bo2, jnp.zeros((5,), jnp.float32)])

    r = _chain(stats, atom_embed, wpos, W_in[3:], b_in.reshape(1, H),
               Wc1, bc1.reshape(1, H), Wc2, bc2.reshape(1, H),
               Wc3, bc3.reshape(1, H), Wo1, bo1.reshape(1, H),
               wo2p, bo2p.reshape(1, 8))

    gidx = (batch_p[:, None] * 8 + jnp.arange(3, dtype=jnp.int32)).reshape(NW, 3 * NPW)
    gidx = jnp.pad(gidx, ((0, 0), (0, 64))).reshape(NW, 8, 128)

    out = _gather_k(r.reshape(RSZ), gidx)
    return out.reshape(N, 3)


# speedup vs baseline: 9.1702x; 1.4575x over previous
import functools

import jax
import jax.numpy as jnp
from jax import lax
from jax.experimental import pallas as pl
from jax.experimental.pallas import tpu as pltpu
from jax.experimental.pallas import tpu_sc as plsc

N = 10000
G = 200
A = 128
H = 128

NW = 32
NPW = 320
NPAD = NW * NPW
NCHUNK = NPW // 16
SECPT = 384
TRASHB = (G + 1) * A
ACCSZ = TRASHB + NW * SECPT
CH = 512
NCH = NPAD // CH

_mesh = plsc.VectorSubcoreMesh(core_axis_name="c", subcore_axis_name="s")


@functools.partial(
    pl.kernel,
    mesh=_mesh,
    out_type=jax.ShapeDtypeStruct((2, ACCSZ), jnp.float32),
    scratch_types=[
        pltpu.VMEM((NPW,), jnp.int32),
        pltpu.VMEM((NPW,), jnp.int32),
        pltpu.VMEM((3, 128), jnp.int32),
        pltpu.VMEM((3, 128), jnp.float32),
        pltpu.VMEM_SHARED((ACCSZ,), jnp.float32),
        pltpu.SemaphoreType.DMA,
        pltpu.SemaphoreType.DMA,
        pltpu.SemaphoreType.DMA,
    ],
)
def _reduce_k(batch_hbm, h_hbm, zeros_hbm, out_hbm,
              bt_v, h_v, idx_st, val_st, acc_sh, sem_in, sem_z, sem_sc):
    c = lax.axis_index("c")
    s = lax.axis_index("s")
    wid = s * 2 + c
    base = wid * NPW

    in_cp = [
        pltpu.async_copy(batch_hbm.at[pl.ds(base, NPW)], bt_v, sem_in),
        pltpu.async_copy(h_hbm.at[pl.ds(base, NPW)], h_v, sem_in),
    ]
    @pl.when(s == 0)
    def _():
        pltpu.async_copy(zeros_hbm, acc_sh, sem_z)
    for cp in in_cp:
        cp.wait()

    lane = lax.iota(jnp.int32, 16)
    ones = jnp.full((16,), 1.0, jnp.float32)
    trash0 = TRASHB + wid * SECPT

    for j in range(NCHUNK):
        sl = pl.ds(j * 16, 16)
        nid = base + j * 16 + lane
        idx = jnp.where(nid < N, bt_v[sl] * A + h_v[sl],
                        trash0 + j * 16 + lane)
        e0 = j * 16
        idx_st[e0 // 128, pl.ds(e0 % 128, 16)] = idx
        val_st[e0 // 128, pl.ds(e0 % 128, 16)] = ones
    for k in range(4):
        e0 = NPW + k * 16
        idx_st[e0 // 128, pl.ds(e0 % 128, 16)] = trash0 + e0 + lane
        val_st[e0 // 128, pl.ds(e0 % 128, 16)] = ones

    @pl.when(s == 0)
    def _():
        pltpu.make_async_copy(zeros_hbm, acc_sh, sem_z).wait()
    plsc.subcore_barrier()

    sc_cp = [pltpu.async_copy(val_st.at[rr], acc_sh.at[idx_st.at[rr]],
                              sem_sc, add=True) for rr in range(3)]
    for cp in sc_cp:
        cp.wait()

    plsc.subcore_barrier()

    @pl.when(s == 0)
    def _():
        pltpu.sync_copy(acc_sh, out_hbm.at[c])


def _chain_body(stats_ref, bat_ref, posp_ref, emb_ref, wpos_ref, winh_ref,
                bin_ref, wc1_ref, bc1_ref, wc2_ref, bc2_ref, wc3_ref,
                bc3_ref, wo1_ref, bo1_ref, wo2_ref, bo2_ref, out_ref):
    f32 = jnp.float32
    st = stats_ref[0] + stats_ref[1]
    hist = st[:G, :]
    cnt = jnp.sum(hist, axis=1, keepdims=True)
    inv = jnp.where(cnt > 0, 1.0 / cnt, 0.0)

    gcol = lax.broadcasted_iota(jnp.int32, (G, 1), 0)
    poss = jnp.zeros((G, 8), f32)
    for j in range(NCH):
        bch = bat_ref[pl.ds(j * CH, CH)]
        oh = jnp.where(bch[None, :] == gcol, 1.0, 0.0).astype(f32)
        poss = poss + jnp.dot(oh, posp_ref[pl.ds(j * CH, CH), :],
                              preferred_element_type=f32)

    wemb = jnp.dot(emb_ref[...], winh_ref[...], preferred_element_type=f32)
    msum = (jnp.dot(hist, wemb, preferred_element_type=f32)
            + jnp.dot(poss, wpos_ref[...], preferred_element_type=f32))
    x = msum * inv + bin_ref[...]
    for w_ref, b_ref in ((wc1_ref, bc1_ref), (wc2_ref, bc2_ref),
                         (wc3_ref, bc3_ref), (wo1_ref, bo1_ref)):
        x = jax.nn.silu(jnp.dot(x, w_ref[...], preferred_element_type=f32)
                        + b_ref[...])
    r = jnp.dot(x, wo2_ref[...], preferred_element_type=f32) + bo2_ref[...]

    grow = lax.broadcasted_iota(jnp.int32, (1, G), 1)
    for j in range(NCH):
        bch = bat_ref[pl.ds(j * CH, CH)]
        oht = jnp.where(bch[:, None] == grow, 1.0, 0.0).astype(f32)
        och = jnp.dot(oht, r, preferred_element_type=f32)
        if (j + 1) * CH <= N:
            out_ref[pl.ds(j * CH, CH), :] = och[:, :3]
        else:
            rem = N - j * CH
            if rem > 0:
                out_ref[pl.ds(j * CH, rem), :] = och[:rem, :3]


_chain = pl.pallas_call(
    _chain_body,
    out_shape=jax.ShapeDtypeStruct((N, 3), jnp.float32),
)


def kernel(pos, h, batch, atom_embed, W_in, b_in, Wc1, bc1, Wc2, bc2,
           Wc3, bc3, Wo1, bo1, Wo2, bo2):
    pad = NPAD - N
    batch_p = jnp.concatenate([batch.astype(jnp.int32),
                               jnp.full((pad,), G, jnp.int32)])
    h_p = jnp.concatenate([h.astype(jnp.int32), jnp.zeros((pad,), jnp.int32)])
    posp = jnp.pad(pos, ((0, pad), (0, 5)))
    zeros = jnp.zeros((ACCSZ,), jnp.float32)

    stats = _reduce_k(batch_p, h_p, zeros)
    stats = stats.reshape(2, ACCSZ // 128, 128)

    wpos = jnp.concatenate([W_in[:3], jnp.zeros((5, H), jnp.float32)])
    wo2p = jnp.concatenate([Wo2, jnp.zeros((H, 5), jnp.float32)], axis=1)
    bo2p = jnp.concatenate([bo2, jnp.zeros((5,), jnp.float32)])

    out = _chain(stats, batch_p, posp, atom_embed, wpos, W_in[3:],
                 b_in.reshape(1, H), Wc1, bc1.reshape(1, H),
                 Wc2, bc2.reshape(1, H), Wc3, bc3.reshape(1, H),
                 Wo1, bo1.reshape(1, H), wo2p, bo2p.reshape(1, 8))
    return out


# speedup vs baseline: 10.7877x; 1.1764x over previous
import functools

import jax
import jax.numpy as jnp
from jax import lax
from jax.experimental import pallas as pl
from jax.experimental.pallas import tpu as pltpu
from jax.experimental.pallas import tpu_sc as plsc

N = 10000
G = 200
A = 128
H = 128

NW = 32
NPW = 320
NCHUNK = NPW // 16
NTAIL = N - (NW - 1) * NPW
SECPT = 384
TRASHB = (G + 1) * A
ACCSZ = TRASHB + NW * SECPT
CH = 512
NCH = (N + CH - 1) // CH

_mesh = plsc.VectorSubcoreMesh(core_axis_name="c", subcore_axis_name="s")


@functools.partial(
    pl.kernel,
    mesh=_mesh,
    out_type=jax.ShapeDtypeStruct((2, ACCSZ), jnp.float32),
    scratch_types=[
        pltpu.VMEM((NPW,), jnp.int32),
        pltpu.VMEM((NPW,), jnp.int32),
        pltpu.VMEM((3, 128), jnp.int32),
        pltpu.VMEM((3, 128), jnp.float32),
        pltpu.VMEM_SHARED((ACCSZ,), jnp.float32),
        pltpu.SemaphoreType.DMA,
        pltpu.SemaphoreType.DMA,
        pltpu.SemaphoreType.DMA,
    ],
)
def _reduce_k(batch_hbm, h_hbm, zeros_hbm, out_hbm,
              bt_v, h_v, idx_st, val_st, acc_sh, sem_in, sem_z, sem_sc):
    c = lax.axis_index("c")
    s = lax.axis_index("s")
    wid = s * 2 + c
    base = wid * NPW

    @pl.when(s == 0)
    def _():
        pltpu.async_copy(zeros_hbm, acc_sh, sem_z)

    @pl.when(wid != NW - 1)
    def _():
        pltpu.sync_copy(batch_hbm.at[pl.ds(base, NPW)], bt_v)
        pltpu.sync_copy(h_hbm.at[pl.ds(base, NPW)], h_v)

    @pl.when(wid == NW - 1)
    def _():
        pltpu.sync_copy(batch_hbm.at[pl.ds((NW - 1) * NPW, NTAIL)],
                        bt_v.at[pl.ds(0, NTAIL)])
        pltpu.sync_copy(h_hbm.at[pl.ds((NW - 1) * NPW, NTAIL)],
                        h_v.at[pl.ds(0, NTAIL)])

    lane = lax.iota(jnp.int32, 16)
    ones = jnp.full((16,), 1.0, jnp.float32)
    trash0 = TRASHB + wid * SECPT

    for j in range(NCHUNK):
        sl = pl.ds(j * 16, 16)
        nid = base + j * 16 + lane
        idx = jnp.where(nid < N, bt_v[sl] * A + h_v[sl],
                        trash0 + j * 16 + lane)
        e0 = j * 16
        idx_st[e0 // 128, pl.ds(e0 % 128, 16)] = idx
        val_st[e0 // 128, pl.ds(e0 % 128, 16)] = ones
    for k in range(4):
        e0 = NPW + k * 16
        idx_st[e0 // 128, pl.ds(e0 % 128, 16)] = trash0 + e0 + lane
        val_st[e0 // 128, pl.ds(e0 % 128, 16)] = ones

    @pl.when(s == 0)
    def _():
        pltpu.make_async_copy(zeros_hbm, acc_sh, sem_z).wait()
    plsc.subcore_barrier()

    sc_cp = [pltpu.async_copy(val_st.at[rr], acc_sh.at[idx_st.at[rr]],
                              sem_sc, add=True) for rr in range(3)]
    for cp in sc_cp:
        cp.wait()

    plsc.subcore_barrier()

    @pl.when(s == 0)
    def _():
        pltpu.sync_copy(acc_sh, out_hbm.at[c])


def _chain_body(stats_ref, bat_ref, pos_ref, emb_ref, win_ref, bin_ref,
                wc1_ref, bc1_ref, wc2_ref, bc2_ref, wc3_ref, bc3_ref,
                wo1_ref, bo1_ref, wo2_ref, bo2_ref, out_ref):
    f32 = jnp.float32
    st = stats_ref[0] + stats_ref[1]
    hist = st[:G, :]
    cnt = jnp.sum(hist, axis=1, keepdims=True)
    inv = jnp.where(cnt > 0, 1.0 / cnt, 0.0)

    gcol = lax.broadcasted_iota(jnp.int32, (G, 1), 0)
    poss = jnp.zeros((G, 3), f32)
    for j in range(NCH):
        n = CH if (j + 1) * CH <= N else N - j * CH
        bch = bat_ref[pl.ds(j * CH, n)]
        oh = jnp.where(bch[None, :] == gcol, 1.0, 0.0).astype(f32)
        poss = poss + jnp.dot(oh, pos_ref[pl.ds(j * CH, n), :],
                              preferred_element_type=f32)

    wemb = jnp.dot(emb_ref[...], win_ref[3:, :], preferred_element_type=f32)
    msum = (jnp.dot(hist, wemb, preferred_element_type=f32)
            + jnp.dot(poss, win_ref[:3, :], preferred_element_type=f32))
    x = msum * inv + bin_ref[...]
    for w_ref, b_ref in ((wc1_ref, bc1_ref), (wc2_ref, bc2_ref),
                         (wc3_ref, bc3_ref), (wo1_ref, bo1_ref)):
        x = jax.nn.silu(jnp.dot(x, w_ref[...], preferred_element_type=f32)
                        + b_ref[...])
    r = jnp.dot(x, wo2_ref[...], preferred_element_type=f32) + bo2_ref[...]

    grow = lax.broadcasted_iota(jnp.int32, (1, G), 1)
    for j in range(NCH):
        n = CH if (j + 1) * CH <= N else N - j * CH
        bch = bat_ref[pl.ds(j * CH, n)]
        oht = jnp.where(bch[:, None] == grow, 1.0, 0.0).astype(f32)
        out_ref[pl.ds(j * CH, n), :] = jnp.dot(oht, r,
                                               preferred_element_type=f32)


_chain = pl.pallas_call(
    _chain_body,
    out_shape=jax.ShapeDtypeStruct((N, 3), jnp.float32),
)


def kernel(pos, h, batch, atom_embed, W_in, b_in, Wc1, bc1, Wc2, bc2,
           Wc3, bc3, Wo1, bo1, Wo2, bo2):
    zeros = jnp.zeros((ACCSZ,), jnp.float32)
    stats = _reduce_k(batch.astype(jnp.int32), h.astype(jnp.int32), zeros)
    stats = stats.reshape(2, ACCSZ // 128, 128)
    return _chain(stats, batch.astype(jnp.int32), pos, atom_embed, W_in,
                  b_in, Wc1, bc1, Wc2, bc2, Wc3, bc3, Wo1, bo1, Wo2, bo2)


# speedup vs baseline: 11.3674x; 1.0537x over previous
import functools

import jax
import jax.numpy as jnp
from jax import lax
from jax.experimental import pallas as pl
from jax.experimental.pallas import tpu as pltpu
from jax.experimental.pallas import tpu_sc as plsc

N = 10000
G = 200
A = 128
H = 128

NW = 32
NPW = 320
NCHUNK = NPW // 16
NTAIL = N - (NW - 1) * NPW
SECPT = 384
TRASHB = G * A
ACCSZ = TRASHB + NW * SECPT
ZSTRIPE = ACCSZ // 16
CH = 512
NCH = (N + CH - 1) // CH

_mesh = plsc.VectorSubcoreMesh(core_axis_name="c", subcore_axis_name="s")


@functools.partial(
    pl.kernel,
    mesh=_mesh,
    out_type=jax.ShapeDtypeStruct((2, ACCSZ), jnp.float32),
    scratch_types=[
        pltpu.VMEM((NPW,), jnp.int32),
        pltpu.VMEM((NPW,), jnp.int32),
        pltpu.VMEM((3, 128), jnp.int32),
        pltpu.VMEM((3, 128), jnp.float32),
        pltpu.VMEM((ZSTRIPE,), jnp.float32),
        pltpu.VMEM_SHARED((ACCSZ,), jnp.float32),
        pltpu.SemaphoreType.DMA,
        pltpu.SemaphoreType.DMA,
    ],
)
def _reduce_k(batch_hbm, h_hbm, out_hbm,
              bt_v, h_v, idx_st, val_st, z_v, acc_sh, sem_in, sem_sc):
    c = lax.axis_index("c")
    s = lax.axis_index("s")
    wid = s * 2 + c
    base = wid * NPW

    @pl.when(wid != NW - 1)
    def _():
        pltpu.async_copy(batch_hbm.at[pl.ds(base, NPW)], bt_v, sem_in)
        pltpu.async_copy(h_hbm.at[pl.ds(base, NPW)], h_v, sem_in)

    @pl.when(wid == NW - 1)
    def _():
        pltpu.async_copy(batch_hbm.at[pl.ds((NW - 1) * NPW, NTAIL)],
                         bt_v.at[pl.ds(0, NTAIL)], sem_in)
        pltpu.async_copy(h_hbm.at[pl.ds((NW - 1) * NPW, NTAIL)],
                         h_v.at[pl.ds(0, NTAIL)], sem_in)

    zero16 = jnp.zeros((16,), jnp.float32)
    for k in range(ZSTRIPE // 16):
        z_v[pl.ds(k * 16, 16)] = zero16
    pltpu.sync_copy(z_v, acc_sh.at[pl.ds(s * ZSTRIPE, ZSTRIPE)])

    lane = lax.iota(jnp.int32, 16)
    ones = jnp.full((16,), 1.0, jnp.float32)
    trash0 = TRASHB + wid * SECPT

    @pl.when(wid != NW - 1)
    def _():
        pltpu.make_async_copy(batch_hbm.at[pl.ds(base, NPW)], bt_v,
                              sem_in).wait()
        pltpu.make_async_copy(h_hbm.at[pl.ds(base, NPW)], h_v, sem_in).wait()

    @pl.when(wid == NW - 1)
    def _():
        pltpu.make_async_copy(batch_hbm.at[pl.ds((NW - 1) * NPW, NTAIL)],
                              bt_v.at[pl.ds(0, NTAIL)], sem_in).wait()
        pltpu.make_async_copy(h_hbm.at[pl.ds((NW - 1) * NPW, NTAIL)],
                              h_v.at[pl.ds(0, NTAIL)], sem_in).wait()

    for j in range(NCHUNK):
        sl = pl.ds(j * 16, 16)
        nid = base + j * 16 + lane
        idx = jnp.where(nid < N, bt_v[sl] * A + h_v[sl],
                        trash0 + j * 16 + lane)
        e0 = j * 16
        idx_st[e0 // 128, pl.ds(e0 % 128, 16)] = idx
        val_st[e0 // 128, pl.ds(e0 % 128, 16)] = ones
    for k in range(4):
        e0 = NPW + k * 16
        idx_st[e0 // 128, pl.ds(e0 % 128, 16)] = trash0 + e0 + lane
        val_st[e0 // 128, pl.ds(e0 % 128, 16)] = ones

    plsc.subcore_barrier()

    sc_cp = [pltpu.async_copy(val_st.at[rr], acc_sh.at[idx_st.at[rr]],
                              sem_sc, add=True) for rr in range(3)]
    for cp in sc_cp:
        cp.wait()

    plsc.subcore_barrier()

    @pl.when(s == 0)
    def _():
        pltpu.sync_copy(acc_sh, out_hbm.at[c])


def _chain_body(stats_ref, bat_ref, pos_ref, emb_ref, win_ref, bin_ref,
                wc1_ref, bc1_ref, wc2_ref, bc2_ref, wc3_ref, bc3_ref,
                wo1_ref, bo1_ref, wo2_ref, bo2_ref, out_ref):
    f32 = jnp.float32
    bf = jnp.bfloat16
    st = stats_ref[0] + stats_ref[1]
    hist = st[:G, :]
    cnt = jnp.sum(hist, axis=1, keepdims=True)
    inv = jnp.where(cnt > 0, 1.0 / cnt, 0.0)

    gcol = lax.broadcasted_iota(jnp.int32, (G, 1), 0)
    poss6 = jnp.zeros((G, 8), f32)
    for j in range(NCH):
        n = CH if (j + 1) * CH <= N else N - j * CH
        bch = bat_ref[pl.ds(j * CH, n)]
        oh = jnp.where(bch[None, :] == gcol, 1.0, 0.0).astype(bf)
        pch = pos_ref[pl.ds(j * CH, n), :]
        phi = pch.astype(bf)
        plo = (pch - phi.astype(f32)).astype(bf)
        phl = jnp.concatenate([phi, plo, jnp.zeros((n, 2), bf)], axis=1)
        poss6 = poss6 + jnp.dot(oh, phl, preferred_element_type=f32)
    poss = poss6[:, :3] + poss6[:, 3:6]

    wemb = jnp.dot(emb_ref[...], win_ref[3:, :], preferred_element_type=f32)
    msum = (jnp.dot(hist, wemb, preferred_element_type=f32)
            + jnp.dot(poss, win_ref[:3, :], preferred_element_type=f32))
    x = msum * inv + bin_ref[...]
    for w_ref, b_ref in ((wc1_ref, bc1_ref), (wc2_ref, bc2_ref),
                         (wc3_ref, bc3_ref), (wo1_ref, bo1_ref)):
        x = jax.nn.silu(jnp.dot(x, w_ref[...], preferred_element_type=f32)
                        + b_ref[...])
    r = jnp.dot(x, wo2_ref[...], preferred_element_type=f32) + bo2_ref[...]

    rhi = r.astype(bf)
    rlo = (r - rhi.astype(f32)).astype(bf)
    rhl = jnp.concatenate([rhi, rlo, jnp.zeros((G, 2), bf)], axis=1)
    grow = lax.broadcasted_iota(jnp.int32, (1, G), 1)
    for j in range(NCH):
        n = CH if (j + 1) * CH <= N else N - j * CH
        bch = bat_ref[pl.ds(j * CH, n)]
        oht = jnp.where(bch[:, None] == grow, 1.0, 0.0).astype(bf)
        och = jnp.dot(oht, rhl, preferred_element_type=f32)
        out_ref[pl.ds(j * CH, n), :] = och[:, :3] + och[:, 3:6]


_chain = pl.pallas_call(
    _chain_body,
    out_shape=jax.ShapeDtypeStruct((N, 3), jnp.float32),
)


def kernel(pos, h, batch, atom_embed, W_in, b_in, Wc1, bc1, Wc2, bc2,
           Wc3, bc3, Wo1, bo1, Wo2, bo2):
    stats = _reduce_k(batch.astype(jnp.int32), h.astype(jnp.int32))
    stats = stats.reshape(2, ACCSZ // 128, 128)
    return _chain(stats, batch.astype(jnp.int32), pos, atom_embed, W_in,
                  b_in, Wc1, bc1, Wc2, bc2, Wc3, bc3, Wo1, bo1, Wo2, bo2)


# speedup vs baseline: 11.5355x; 1.0148x over previous
import functools

import jax
import jax.numpy as jnp
from jax import lax
from jax.experimental import pallas as pl
from jax.experimental.pallas import tpu as pltpu
from jax.experimental.pallas import tpu_sc as plsc

N = 10000
G = 200
A = 128
H = 128

NW = 32
NPW = 320
NTAIL = N - (NW - 1) * NPW
SECPT = 384
TRASHB = G * A
ACCSZ = TRASHB + NW * SECPT
ZSTRIPE = TRASHB // 16

_mesh = plsc.VectorSubcoreMesh(core_axis_name="c", subcore_axis_name="s")


@functools.partial(
    pl.kernel,
    mesh=_mesh,
    out_type=jax.ShapeDtypeStruct((2, ACCSZ), jnp.float32),
    scratch_types=[
        pltpu.VMEM((SECPT,), jnp.int32),
        pltpu.VMEM((SECPT,), jnp.int32),
        pltpu.VMEM((3, 128), jnp.int32),
        pltpu.VMEM((3, 128), jnp.float32),
        pltpu.VMEM((ZSTRIPE,), jnp.float32),
        pltpu.VMEM_SHARED((ACCSZ,), jnp.float32),
        pltpu.SemaphoreType.DMA,
        pltpu.SemaphoreType.DMA,
    ],
)
def _reduce_k(batch_hbm, h_hbm, out_hbm,
              bt_v, h_v, idx_st, val_st, z_v, acc_sh, sem_in, sem_sc):
    c = lax.axis_index("c")
    s = lax.axis_index("s")
    wid = s * 2 + c
    base = wid * NPW

    @pl.when(wid != NW - 1)
    def _():
        pltpu.async_copy(batch_hbm.at[pl.ds(base, NPW)],
                         bt_v.at[pl.ds(0, NPW)], sem_in)
        pltpu.async_copy(h_hbm.at[pl.ds(base, NPW)],
                         h_v.at[pl.ds(0, NPW)], sem_in)

    @pl.when(wid == NW - 1)
    def _():
        pltpu.async_copy(batch_hbm.at[pl.ds((NW - 1) * NPW, NTAIL)],
                         bt_v.at[pl.ds(0, NTAIL)], sem_in)
        pltpu.async_copy(h_hbm.at[pl.ds((NW - 1) * NPW, NTAIL)],
                         h_v.at[pl.ds(0, NTAIL)], sem_in)

    zero16 = jnp.zeros((16,), jnp.float32)

    def _zb(k, _):
        z_v[pl.ds(k * 16, 16)] = zero16
        return _
    lax.fori_loop(0, ZSTRIPE // 16, _zb, None)
    pltpu.sync_copy(z_v, acc_sh.at[pl.ds(s * ZSTRIPE, ZSTRIPE)])

    lane = lax.iota(jnp.int32, 16)
    ones = jnp.full((16,), 1.0, jnp.float32)
    trash0 = TRASHB + wid * SECPT

    @pl.when(wid != NW - 1)
    def _():
        pltpu.make_async_copy(batch_hbm.at[pl.ds(base, NPW)],
                              bt_v.at[pl.ds(0, NPW)], sem_in).wait()
        pltpu.make_async_copy(h_hbm.at[pl.ds(base, NPW)],
                              h_v.at[pl.ds(0, NPW)], sem_in).wait()

    @pl.when(wid == NW - 1)
    def _():
        pltpu.make_async_copy(batch_hbm.at[pl.ds((NW - 1) * NPW, NTAIL)],
                              bt_v.at[pl.ds(0, NTAIL)], sem_in).wait()
        pltpu.make_async_copy(h_hbm.at[pl.ds((NW - 1) * NPW, NTAIL)],
                              h_v.at[pl.ds(0, NTAIL)], sem_in).wait()

    for r in range(3):
        def _sb(k, _, r=r):
            ent = r * 128 + k * 16
            sl = pl.ds(ent, 16)
            nid = base + ent + lane
            ok = jnp.logical_and(nid < N, ent + lane < NPW)
            idx = jnp.where(ok, bt_v[sl] * A + h_v[sl], trash0 + ent + lane)
            col = pl.ds(k * 16, 16)
            idx_st[r, col] = idx
            val_st[r, col] = ones
            return _
        lax.fori_loop(0, 8, _sb, None)

    plsc.subcore_barrier()

    sc_cp = [pltpu.async_copy(val_st.at[rr], acc_sh.at[idx_st.at[rr]],
                              sem_sc, add=True) for rr in range(3)]
    for cp in sc_cp:
        cp.wait()

    plsc.subcore_barrier()

    @pl.when(s == 0)
    def _():
        pltpu.sync_copy(acc_sh, out_hbm.at[c])


def _chain_body(stats_ref, bat_ref, pos_ref, emb_ref, win_ref, bin_ref,
                wc1_ref, bc1_ref, wc2_ref, bc2_ref, wc3_ref, bc3_ref,
                wo1_ref, bo1_ref, wo2_ref, bo2_ref, out_ref):
    f32 = jnp.float32
    bf = jnp.bfloat16
    st = stats_ref[0] + stats_ref[1]
    hist = st[:G, :]
    cnt = jnp.sum(hist, axis=1, keepdims=True)
    inv = jnp.where(cnt > 0, 1.0 / cnt, 0.0)

    one_b = jnp.bfloat16(1.0)
    zero_b = jnp.bfloat16(0.0)
    bat_b = bat_ref[...].astype(bf)
    gcol = lax.broadcasted_iota(jnp.int32, (G, 1), 0).astype(bf)
    oh = jnp.where(bat_b[None, :] == gcol, one_b, zero_b)

    pch = pos_ref[...]
    phi = pch.astype(bf)
    plo = (pch - phi.astype(f32)).astype(bf)
    phl = jnp.concatenate([phi, plo, jnp.zeros((N, 2), bf)], axis=1)
    poss6 = jnp.dot(oh, phl, preferred_element_type=f32)
    poss = poss6[:, :3] + poss6[:, 3:6]

    wemb = jnp.dot(emb_ref[...], win_ref[3:, :], preferred_element_type=f32)
    msum = (jnp.dot(hist, wemb, preferred_element_type=f32)
            + jnp.dot(poss, win_ref[:3, :], preferred_element_type=f32))
    x = msum * inv + bin_ref[...]
    for w_ref, b_ref in ((wc1_ref, bc1_ref), (wc2_ref, bc2_ref),
                         (wc3_ref, bc3_ref), (wo1_ref, bo1_ref)):
        x = jax.nn.silu(jnp.dot(x, w_ref[...], preferred_element_type=f32)
                        + b_ref[...])
    r = jnp.dot(x, wo2_ref[...], preferred_element_type=f32) + bo2_ref[...]

    rhi = r.astype(bf)
    rlo = (r - rhi.astype(f32)).astype(bf)
    rhl = jnp.concatenate([rhi, rlo, jnp.zeros((G, 2), bf)], axis=1)
    grow = lax.broadcasted_iota(jnp.int32, (1, G), 1).astype(bf)
    oht = jnp.where(bat_b[:, None] == grow, one_b, zero_b)
    och = jnp.dot(oht, rhl, preferred_element_type=f32)
    out_ref[...] = och[:, :3] + och[:, 3:6]


_chain = pl.pallas_call(
    _chain_body,
    out_shape=jax.ShapeDtypeStruct((N, 3), jnp.float32),
)


def kernel(pos, h, batch, atom_embed, W_in, b_in, Wc1, bc1, Wc2, bc2,
           Wc3, bc3, Wo1, bo1, Wo2, bo2):
    stats = _reduce_k(batch.astype(jnp.int32), h.astype(jnp.int32))
    stats = stats.reshape(2, ACCSZ // 128, 128)
    return _chain(stats, batch.astype(jnp.int32), pos, atom_embed, W_in,
                  b_in, Wc1, bc1, Wc2, bc2, Wc3, bc3, Wo1, bo1, Wo2, bo2)


# speedup vs baseline: 14.0095x; 1.2145x over previous
import functools

import jax
import jax.numpy as jnp
from jax import lax
from jax.experimental import pallas as pl
from jax.experimental.pallas import tpu as pltpu
from jax.experimental.pallas import tpu_sc as plsc

N = 10000
G = 200
A = 128
H = 128

NW = 32
NPW = 320
NTAIL = N - (NW - 1) * NPW
SECPT = 384
TRASHB = G * A
ACCSZ = TRASHB + NW * SECPT
ZSTRIPE = TRASHB // 16

_mesh = plsc.VectorSubcoreMesh(core_axis_name="c", subcore_axis_name="s")


@functools.partial(
    pl.kernel,
    mesh=_mesh,
    out_type=jax.ShapeDtypeStruct((2, ACCSZ), jnp.float32),
    scratch_types=[
        pltpu.VMEM((SECPT,), jnp.int32),
        pltpu.VMEM((SECPT,), jnp.int32),
        pltpu.VMEM((3, 128), jnp.int32),
        pltpu.VMEM((3, 128), jnp.float32),
        pltpu.VMEM((ZSTRIPE,), jnp.float32),
        pltpu.VMEM_SHARED((ACCSZ,), jnp.float32),
        pltpu.SemaphoreType.DMA,
        pltpu.SemaphoreType.DMA,
    ],
)
def _reduce_k(batch_hbm, h_hbm, out_hbm,
              bt_v, h_v, idx_st, val_st, z_v, acc_sh, sem_in, sem_sc):
    c = lax.axis_index("c")
    s = lax.axis_index("s")
    wid = s * 2 + c
    base = wid * NPW

    @pl.when(wid != NW - 1)
    def _():
        pltpu.async_copy(batch_hbm.at[pl.ds(base, NPW)],
                         bt_v.at[pl.ds(0, NPW)], sem_in)
        pltpu.async_copy(h_hbm.at[pl.ds(base, NPW)],
                         h_v.at[pl.ds(0, NPW)], sem_in)

    @pl.when(wid == NW - 1)
    def _():
        pltpu.async_copy(batch_hbm.at[pl.ds((NW - 1) * NPW, NTAIL)],
                         bt_v.at[pl.ds(0, NTAIL)], sem_in)
        pltpu.async_copy(h_hbm.at[pl.ds((NW - 1) * NPW, NTAIL)],
                         h_v.at[pl.ds(0, NTAIL)], sem_in)

    zero16 = jnp.zeros((16,), jnp.float32)

    def _zb(k, _):
        z_v[pl.ds(k * 16, 16)] = zero16
        return _
    lax.fori_loop(0, ZSTRIPE // 16, _zb, None)
    pltpu.sync_copy(z_v, acc_sh.at[pl.ds(s * ZSTRIPE, ZSTRIPE)])

    lane = lax.iota(jnp.int32, 16)
    ones = jnp.full((16,), 1.0, jnp.float32)
    trash0 = TRASHB + wid * SECPT

    @pl.when(wid != NW - 1)
    def _():
        pltpu.make_async_copy(batch_hbm.at[pl.ds(base, NPW)],
                              bt_v.at[pl.ds(0, NPW)], sem_in).wait()
        pltpu.make_async_copy(h_hbm.at[pl.ds(base, NPW)],
                              h_v.at[pl.ds(0, NPW)], sem_in).wait()

    @pl.when(wid == NW - 1)
    def _():
        pltpu.make_async_copy(batch_hbm.at[pl.ds((NW - 1) * NPW, NTAIL)],
                              bt_v.at[pl.ds(0, NTAIL)], sem_in).wait()
        pltpu.make_async_copy(h_hbm.at[pl.ds((NW - 1) * NPW, NTAIL)],
                              h_v.at[pl.ds(0, NTAIL)], sem_in).wait()

    for r in range(3):
        def _sb(k, _, r=r):
            ent = r * 128 + k * 16
            sl = pl.ds(ent, 16)
            nid = base + ent + lane
            ok = jnp.logical_and(nid < N, ent + lane < NPW)
            idx = jnp.where(ok, bt_v[sl] * A + h_v[sl], trash0 + ent + lane)
            col = pl.ds(k * 16, 16)
            idx_st[r, col] = idx
            val_st[r, col] = ones
            return _
        lax.fori_loop(0, 8, _sb, None)

    plsc.subcore_barrier()

    sc_cp = [pltpu.async_copy(val_st.at[rr], acc_sh.at[idx_st.at[rr]],
                              sem_sc, add=True) for rr in range(3)]
    for cp in sc_cp:
        cp.wait()

    plsc.subcore_barrier()

    @pl.when(s == 0)
    def _():
        pltpu.sync_copy(acc_sh, out_hbm.at[c])


def _chain_body(stats_ref, bat_ref, pos_ref, emb_ref, win_ref, bin_ref,
                wc1_ref, bc1_ref, wc2_ref, bc2_ref, wc3_ref, bc3_ref,
                wo1_ref, bo1_ref, wo2_ref, bo2_ref, out_ref):
    f32 = jnp.float32
    bf = jnp.bfloat16
    st = stats_ref[0] + stats_ref[1]
    hist = st[:G, :]
    cnt = jnp.sum(hist, axis=1, keepdims=True)
    inv = jnp.where(cnt > 0, 1.0 / cnt, 0.0)

    one_b = jnp.bfloat16(1.0)
    zero_b = jnp.bfloat16(0.0)
    bat_b = bat_ref[...].astype(bf)
    gcol = lax.broadcasted_iota(jnp.int32, (G, 1), 0).astype(bf)
    oh = jnp.where(bat_b[None, :] == gcol, one_b, zero_b)

    pch = pos_ref[...]
    phi = pch.astype(bf)
    plo = (pch - phi.astype(f32)).astype(bf)
    phl = jnp.concatenate([phi, plo, jnp.zeros((N, 2), bf)], axis=1)
    poss6 = jnp.dot(oh, phl, preferred_element_type=f32)
    poss = poss6[:, :3] + poss6[:, 3:6]

    wemb = jnp.dot(emb_ref[...], win_ref[3:, :], preferred_element_type=f32)
    msum = (jnp.dot(hist, wemb, preferred_element_type=f32)
            + jnp.dot(poss, win_ref[:3, :], preferred_element_type=f32))
    x = msum * inv + bin_ref[...]
    for w_ref, b_ref in ((wc1_ref, bc1_ref), (wc2_ref, bc2_ref),
                         (wc3_ref, bc3_ref), (wo1_ref, bo1_ref)):
        x = jax.nn.silu(jnp.dot(x, w_ref[...], preferred_element_type=f32)
                        + b_ref[...])
    r = jnp.dot(x, wo2_ref[...], preferred_element_type=f32) + bo2_ref[...]

    rhi = r.astype(bf)
    rlo = (r - rhi.astype(f32)).astype(bf)
    rhl = jnp.concatenate([rhi, rlo, jnp.zeros((G, 2), bf)], axis=1)
    och = lax.dot_general(rhl, oh, (((0,), (0,)), ((), ())),
                          preferred_element_type=f32)
    out_ref[...] = och[:3, :] + och[3:6, :]


_chain = pl.pallas_call(
    _chain_body,
    out_shape=jax.ShapeDtypeStruct((3, N), jnp.float32),
)


def kernel(pos, h, batch, atom_embed, W_in, b_in, Wc1, bc1, Wc2, bc2,
           Wc3, bc3, Wo1, bo1, Wo2, bo2):
    stats = _reduce_k(batch.astype(jnp.int32), h.astype(jnp.int32))
    stats = stats.reshape(2, ACCSZ // 128, 128)
    outt = _chain(stats, batch.astype(jnp.int32), pos, atom_embed, W_in,
                  b_in, Wc1, bc1, Wc2, bc2, Wc3, bc3, Wo1, bo1, Wo2, bo2)
    return outt.T
